# baseline (speedup 1.0000x reference)
import jax
import jax.numpy as jnp
from jax.experimental import pallas as pl

N = 10000
COMMIT_W = 0.25


def _l2norm(t):
    return t / (jnp.linalg.norm(t, axis=-1, keepdims=True) + 1e-12)


def _graph_conv(x, src, dst, W, b):
    deg_out = jnp.clip(jnp.bincount(src, length=N).astype(x.dtype), 1.0)
    deg_in = jnp.clip(jnp.bincount(dst, length=N).astype(x.dtype), 1.0)
    h = x * (deg_out ** -0.5)[:, None]
    msg = jnp.take(h, src, axis=0)
    agg = jax.ops.segment_sum(msg, dst, num_segments=N)
    agg = agg * (deg_in ** -0.5)[:, None]
    return agg @ W + b


def _vq(h, codebook):
    hn = _l2norm(h)
    cbn = _l2norm(codebook)
    sim = hn @ cbn.T
    idx = jnp.argmax(sim, axis=-1)
    q = jnp.take(cbn, idx, axis=0)
    commit = COMMIT_W * jnp.mean((q - h) ** 2)
    return q, idx, commit


def _id_kernel(x_ref, o_ref):
    o_ref[...] = x_ref[...]


def kernel(feats, edge_index, W1, b1, W2, b2, cb1, cb2,
           p1w1, p1b1, p1w2, p1b2, p2w1, p2b1, p2w2, p2b2):
    src = edge_index[0]
    dst = edge_index[1]
    h = _graph_conv(feats, src, dst, W1, b1)
    h = jax.nn.relu(h)
    h = _graph_conv(h, src, dst, W2, b2)
    q1, _, c1 = _vq(h, cb1)
    q2, _, c2 = _vq(h, cb2)
    proj1 = jax.nn.relu(q1 @ p1w1 + p1b1) @ p1w2 + p1b2
    proj2 = jax.nn.relu(q2 @ p2w1 + p2b1) @ p2w2 + p2b2
    proj1 = pl.pallas_call(
        _id_kernel, out_shape=jax.ShapeDtypeStruct(proj1.shape, proj1.dtype)
    )(proj1)
    return (proj1, proj2, c1, c2)


# speedup vs baseline: 2.6153x; 2.6153x over previous
import functools

import jax
import jax.numpy as jnp
from jax import lax
from jax.experimental import pallas as pl
from jax.experimental.pallas import tpu as pltpu
from jax.experimental.pallas import tpu_sc as plsc

N = 10000
E = 320000
D_IN = 128
D_HID = 256
D_OUT = 256
CB1 = 512
CB2 = 1024
PH_O = 64
COMMIT_W = 0.25

NC = 2
NS = 16
NW = NC * NS
N_PAD = 10240
SLAB = N_PAD // NS
E_PER_W = E // NW
W_FULL = 128
N_WIN = E_PER_W // W_FULL
W_TAIL = E_PER_W - N_WIN * W_FULL

@functools.lru_cache(maxsize=None)
def _mesh():
    return plsc.VectorSubcoreMesh(core_axis_name="c", subcore_axis_name="s",
                                  num_cores=NC, num_subcores=NS)


def _wid(c, s):
    return s * NC + c


def _deg_body(src_hbm, dst_hbm, zeros1, ones_hbm, out, src_v, dst_v, ones_v,
              src_t, dst_t, ones_t, acc_o, acc_i):
    c = lax.axis_index("c")
    s = lax.axis_index("s")
    w = _wid(c, s)
    pltpu.sync_copy(zeros1.at[pl.ds(s * SLAB, SLAB)], acc_o.at[pl.ds(s * SLAB, SLAB)])
    pltpu.sync_copy(zeros1.at[pl.ds(s * SLAB, SLAB)], acc_i.at[pl.ds(s * SLAB, SLAB)])
    pltpu.sync_copy(ones_hbm, ones_v)
    pltpu.sync_copy(ones_hbm.at[pl.ds(0, W_TAIL)], ones_t)
    plsc.subcore_barrier()

    e0 = w * E_PER_W

    def body(i, carry):
        base = e0 + i * W_FULL
        pltpu.sync_copy(src_hbm.at[pl.ds(base, W_FULL)], src_v)
        pltpu.sync_copy(dst_hbm.at[pl.ds(base, W_FULL)], dst_v)
        pltpu.sync_copy(ones_v, acc_o.at[src_v], add=True)
        pltpu.sync_copy(ones_v, acc_i.at[dst_v], add=True)
        return carry

    lax.fori_loop(0, N_WIN, body, 0)
    base = e0 + N_WIN * W_FULL
    pltpu.sync_copy(src_hbm.at[pl.ds(base, W_TAIL)], src_t)
    pltpu.sync_copy(dst_hbm.at[pl.ds(base, W_TAIL)], dst_t)
    pltpu.sync_copy(ones_t, acc_o.at[src_t], add=True)
    pltpu.sync_copy(ones_t, acc_i.at[dst_t], add=True)

    plsc.subcore_barrier()
    pltpu.sync_copy(acc_o.at[pl.ds(s * SLAB, SLAB)], out.at[c, 0, pl.ds(s * SLAB, SLAB)])
    pltpu.sync_copy(acc_i.at[pl.ds(s * SLAB, SLAB)], out.at[c, 1, pl.ds(s * SLAB, SLAB)])


@functools.lru_cache(maxsize=None)
def _deg_kernel():
    return pl.kernel(
        _deg_body,
        out_type=jax.ShapeDtypeStruct((NC, 2, N_PAD), jnp.float32),
        mesh=_mesh(),
        scratch_types=[
            pltpu.VMEM((W_FULL,), jnp.int32),
            pltpu.VMEM((W_FULL,), jnp.int32),
            pltpu.VMEM((W_FULL,), jnp.float32),
            pltpu.VMEM((W_TAIL,), jnp.int32),
            pltpu.VMEM((W_TAIL,), jnp.int32),
            pltpu.VMEM((W_TAIL,), jnp.float32),
            pltpu.VMEM_SHARED((N_PAD,), jnp.float32),
            pltpu.VMEM_SHARED((N_PAD,), jnp.float32),
        ],
    )


def _deg_call(src, dst, zeros1, ones):
    return _deg_kernel()(src, dst, zeros1, ones)


def _agg_body(table, src_hbm, dst_hbm, zeros2, out, src_v, dst_v, src_t,
              dst_t, rows_v, rows_t, acc, sem):
    c = lax.axis_index("c")
    s = lax.axis_index("s")
    w = _wid(c, s)
    pltpu.sync_copy(zeros2.at[pl.ds(s * SLAB, SLAB)], acc.at[pl.ds(s * SLAB, SLAB)])
    plsc.subcore_barrier()

    e0 = w * E_PER_W

    def body(i, carry):
        base = e0 + i * W_FULL
        pltpu.sync_copy(src_hbm.at[pl.ds(base, W_FULL)], src_v)
        pltpu.sync_copy(dst_hbm.at[pl.ds(base, W_FULL)], dst_v)
        pltpu.async_copy(table.at[src_v], rows_v, sem).wait()
        pltpu.sync_copy(rows_v, acc.at[dst_v], add=True)
        return carry

    lax.fori_loop(0, N_WIN, body, 0)
    base = e0 + N_WIN * W_FULL
    pltpu.sync_copy(src_hbm.at[pl.ds(base, W_TAIL)], src_t)
    pltpu.sync_copy(dst_hbm.at[pl.ds(base, W_TAIL)], dst_t)
    pltpu.async_copy(table.at[src_t], rows_t, sem).wait()
    pltpu.sync_copy(rows_t, acc.at[dst_t], add=True)

    plsc.subcore_barrier()
    pltpu.sync_copy(acc.at[pl.ds(s * SLAB, SLAB)], out.at[c, pl.ds(s * SLAB, SLAB)])


@functools.lru_cache(maxsize=None)
def _agg_kernel():
    return pl.kernel(
        _agg_body,
        out_type=jax.ShapeDtypeStruct((NC, N_PAD, D_IN), jnp.float32),
        mesh=_mesh(),
        scratch_types=[
            pltpu.VMEM((W_FULL,), jnp.int32),
            pltpu.VMEM((W_FULL,), jnp.int32),
            pltpu.VMEM((W_TAIL,), jnp.int32),
            pltpu.VMEM((W_TAIL,), jnp.int32),
            pltpu.VMEM((W_FULL, D_IN), jnp.float32),
            pltpu.VMEM((W_TAIL, D_IN), jnp.float32),
            pltpu.VMEM_SHARED((N_PAD, D_IN), jnp.float32),
            pltpu.SemaphoreType.DMA,
        ],
    )


def _agg_call(table, src, dst, zeros2):
    return _agg_kernel()(table, src, dst, zeros2)


_R_PER_W = N_PAD // NW
_GW = 80


def _prj_body(t1, t2, i1, i2, o1, o2, idx_v, rows_v, sem):
    c = lax.axis_index("c")
    s = lax.axis_index("s")
    w = _wid(c, s)
    r0 = w * _R_PER_W

    def body(i, carry):
        base = r0 + i * _GW
        pltpu.sync_copy(i1.at[pl.ds(base, _GW)], idx_v)
        pltpu.async_copy(t1.at[idx_v], rows_v, sem).wait()
        pltpu.sync_copy(rows_v, o1.at[pl.ds(base, _GW)])
        pltpu.sync_copy(i2.at[pl.ds(base, _GW)], idx_v)
        pltpu.async_copy(t2.at[idx_v], rows_v, sem).wait()
        pltpu.sync_copy(rows_v, o2.at[pl.ds(base, _GW)])
        return carry

    lax.fori_loop(0, _R_PER_W // _GW, body, 0)


@functools.lru_cache(maxsize=None)
def _prj_kernel():
    return pl.kernel(
        _prj_body,
        out_type=[jax.ShapeDtypeStruct((N_PAD, D_IN), jnp.float32),
                  jax.ShapeDtypeStruct((N_PAD, D_IN), jnp.float32)],
        mesh=_mesh(),
        scratch_types=[
            pltpu.VMEM((_GW,), jnp.int32),
            pltpu.VMEM((_GW, D_IN), jnp.float32),
            pltpu.SemaphoreType.DMA,
        ],
    )


def _prj_call(t1, t2, i1, i2):
    return _prj_kernel()(t1, t2, i1, i2)


_BLK = 1024
_NB = N_PAD // _BLK


def _mlp1_kernel(p_ref, si_ref, so_ref, w1_ref, b1_ref, lo_ref, hi_ref):
    a = (p_ref[0] + p_ref[1]) * si_ref[...]
    h = jnp.maximum(
        jax.lax.dot_general(a, w1_ref[...], (((1,), (0,)), ((), ())),
                            preferred_element_type=jnp.float32) + b1_ref[...],
        0.0)
    hs = h * so_ref[...]
    lo_ref[...] = hs[:, :D_IN]
    hi_ref[...] = hs[:, D_IN:]


def _mlp1(parts, s_in, s_out, W1, b1):
    return pl.pallas_call(
        _mlp1_kernel,
        grid=(_NB,),
        in_specs=[
            pl.BlockSpec((NC, _BLK, D_IN), lambda i: (0, i, 0)),
            pl.BlockSpec((_BLK, 1), lambda i: (i, 0)),
            pl.BlockSpec((_BLK, 1), lambda i: (i, 0)),
            pl.BlockSpec((D_IN, D_HID), lambda i: (0, 0)),
            pl.BlockSpec((1, D_HID), lambda i: (0, 0)),
        ],
        out_specs=[
            pl.BlockSpec((_BLK, D_IN), lambda i: (i, 0)),
            pl.BlockSpec((_BLK, D_IN), lambda i: (i, 0)),
        ],
        out_shape=[jax.ShapeDtypeStruct((N_PAD, D_IN), jnp.float32),
                   jax.ShapeDtypeStruct((N_PAD, D_IN), jnp.float32)],
    )(parts, s_in, s_out, W1, b1)


def _cbt_kernel(cbn1_ref, cbn2_ref, w11, b11, w12, b12, w21, b21, w22, b22,
                t1_ref, t2_ref):
    cbn1 = cbn1_ref[...]
    cbn2 = cbn2_ref[...]

    def mlp(q, wa, ba, wb, bb):
        h = jnp.maximum(
            jax.lax.dot_general(q, wa[...], (((1,), (0,)), ((), ())),
                                preferred_element_type=jnp.float32) + ba[...], 0.0)
        return (jax.lax.dot_general(h, wb[...], (((1,), (0,)), ((), ())),
                                    preferred_element_type=jnp.float32) + bb[...])

    z1 = jnp.zeros((CB1, D_IN - PH_O), jnp.float32)
    z2 = jnp.zeros((CB2, D_IN - PH_O), jnp.float32)
    t1_ref[...] = jnp.concatenate([mlp(cbn1, w11, b11, w12, b12), z1], axis=1)
    t2_ref[...] = jnp.concatenate([mlp(cbn2, w21, b21, w22, b22), z2], axis=1)


def _cb_tables(cbn1, cbn2, p1w1, p1b1, p1w2, p1b2, p2w1, p2b1, p2w2, p2b2):
    return pl.pallas_call(
        _cbt_kernel,
        out_shape=[jax.ShapeDtypeStruct((CB1, D_IN), jnp.float32),
                   jax.ShapeDtypeStruct((CB2, D_IN), jnp.float32)],
    )(cbn1, cbn2, p1w1, p1b1.reshape(1, -1), p1w2, p1b2.reshape(1, -1),
      p2w1, p2b1.reshape(1, -1), p2w2, p2b2.reshape(1, -1))


def _h2_kernel(lo_p, hi_p, si_ref, w2_ref, b2_ref, h2_ref):
    lo = (lo_p[0] + lo_p[1]) * si_ref[...]
    hi = (hi_p[0] + hi_p[1]) * si_ref[...]
    w2 = w2_ref[...]
    h2_ref[...] = (jax.lax.dot_general(lo, w2[:D_IN], (((1,), (0,)), ((), ())),
                                       preferred_element_type=jnp.float32)
                   + jax.lax.dot_general(hi, w2[D_IN:], (((1,), (0,)), ((), ())),
                                         preferred_element_type=jnp.float32)
                   + b2_ref[...])


def _h2_stage(lo_parts, hi_parts, s_in, W2, b2):
    return pl.pallas_call(
        _h2_kernel,
        grid=(_NB,),
        in_specs=[
            pl.BlockSpec((NC, _BLK, D_IN), lambda i: (0, i, 0)),
            pl.BlockSpec((NC, _BLK, D_IN), lambda i: (0, i, 0)),
            pl.BlockSpec((_BLK, 1), lambda i: (i, 0)),
            pl.BlockSpec((D_OUT, D_OUT), lambda i: (0, 0)),
            pl.BlockSpec((1, D_OUT), lambda i: (0, 0)),
        ],
        out_specs=pl.BlockSpec((_BLK, D_OUT), lambda i: (i, 0)),
        out_shape=jax.ShapeDtypeStruct((N_PAD, D_OUT), jnp.float32),
    )(lo_parts, hi_parts, s_in, W2, b2)


def _vq_kernel(hn_ref, nv_ref, cbn1_ref, cbn2_ref,
               i1_ref, i2_ref, c1_ref, c2_ref):
    i = pl.program_id(0)
    hn = hn_ref[...]
    nr_col = nv_ref[...]

    def vq(cbn, ncb):
        sim = jax.lax.dot_general(hn, cbn, (((1,), (1,)), ((), ())),
                                  preferred_element_type=jnp.float32)
        m = jnp.max(sim, axis=1, keepdims=True)
        col = jax.lax.broadcasted_iota(jnp.int32, sim.shape, 1)
        idx = jnp.min(jnp.where(sim == m, col, ncb), axis=1)
        return m[:, 0], idx

    m1, idx1 = vq(cbn1_ref[...], CB1)
    m2, idx2 = vq(cbn2_ref[...], CB2)
    i1_ref[...] = idx1.reshape(1, 1, _BLK)
    i2_ref[...] = idx2.reshape(1, 1, _BLK)

    row = jax.lax.broadcasted_iota(jnp.int32, (_BLK,), 0) + i * _BLK
    valid = row < N
    nr = nr_col[:, 0]
    ss = nr * nr
    part1 = jnp.sum(jnp.where(valid, 1.0 - 2.0 * m1 * nr + ss, 0.0))
    part2 = jnp.sum(jnp.where(valid, 1.0 - 2.0 * m2 * nr + ss, 0.0))

    @pl.when(i == 0)
    def _():
        c1_ref[...] = jnp.zeros_like(c1_ref)
        c2_ref[...] = jnp.zeros_like(c2_ref)

    c1_ref[...] = c1_ref[...] + part1
    c2_ref[...] = c2_ref[...] + part2


def _vq_stage(hn, nv, cbn1, cbn2):
    return pl.pallas_call(
        _vq_kernel,
        grid=(_NB,),
        in_specs=[
            pl.BlockSpec((_BLK, D_OUT), lambda i: (i, 0)),
            pl.BlockSpec((_BLK, 1), lambda i: (i, 0)),
            pl.BlockSpec((CB1, D_OUT), lambda i: (0, 0)),
            pl.BlockSpec((CB2, D_OUT), lambda i: (0, 0)),
        ],
        out_specs=[
            pl.BlockSpec((1, 1, _BLK), lambda i: (i, 0, 0)),
            pl.BlockSpec((1, 1, _BLK), lambda i: (i, 0, 0)),
            pl.BlockSpec((1, 1), lambda i: (0, 0)),
            pl.BlockSpec((1, 1), lambda i: (0, 0)),
        ],
        out_shape=[jax.ShapeDtypeStruct((_NB, 1, _BLK), jnp.int32),
                   jax.ShapeDtypeStruct((_NB, 1, _BLK), jnp.int32),
                   jax.ShapeDtypeStruct((1, 1), jnp.float32),
                   jax.ShapeDtypeStruct((1, 1), jnp.float32)],
    )(hn, nv, cbn1, cbn2)


def kernel(feats, edge_index, W1, b1, W2, b2, cb1, cb2,
           p1w1, p1b1, p1w2, p1b2, p2w1, p2b1, p2w2, p2b2):
    zeros1 = jnp.zeros((N_PAD,), jnp.float32)
    zeros2 = jnp.zeros((N_PAD, D_IN), jnp.float32)
    ones = jnp.ones((W_FULL,), jnp.float32)

    src = edge_index[0]
    dst = edge_index[1]
    deg = _deg_call(src, dst, zeros1, ones)
    perm = jnp.argsort(dst, stable=True)
    ssrc = jnp.take(src, perm)
    sdst = jnp.take(dst, perm)
    deg_out = jnp.clip(deg[0, 0] + deg[1, 0], 1.0)
    deg_in = jnp.clip(deg[0, 1] + deg[1, 1], 1.0)
    s_out = deg_out ** -0.5
    s_in = deg_in ** -0.5

    cbn1 = cb1 / (jnp.linalg.norm(cb1, axis=-1, keepdims=True) + 1e-12)
    cbn2 = cb2 / (jnp.linalg.norm(cb2, axis=-1, keepdims=True) + 1e-12)
    t1, t2 = _cb_tables(cbn1, cbn2, p1w1, p1b1, p1w2, p1b2,
                        p2w1, p2b1, p2w2, p2b2)

    h0 = feats * s_out[:N, None]
    agg1 = _agg_call(h0, ssrc, sdst, zeros2)

    h1s = _mlp1(agg1, s_in[:, None], s_out[:, None], W1, b1.reshape(1, -1))
    lo_parts = _agg_call(h1s[0], ssrc, sdst, zeros2)
    hi_parts = _agg_call(h1s[1], ssrc, sdst, zeros2)

    h2 = _h2_stage(lo_parts, hi_parts, s_in[:, None], W2, b2.reshape(1, -1))
    nv = jnp.linalg.norm(h2, axis=-1, keepdims=True) + 1e-12
    hn = h2 / nv
    i1, i2, c1s, c2s = _vq_stage(hn, nv, cbn1, cbn2)

    o1, o2 = _prj_call(t1, t2, i1.reshape(N_PAD), i2.reshape(N_PAD))

    scale = COMMIT_W / (N * D_OUT)
    c1 = (c1s[0, 0] * scale).astype(jnp.float32)
    c2 = (c2s[0, 0] * scale).astype(jnp.float32)
    return (o1[:N, :PH_O], o2[:N, :PH_O], c1, c2)


# speedup vs baseline: 2.8389x; 1.0855x over previous
import functools

import jax
import jax.numpy as jnp
from jax import lax
from jax.experimental import pallas as pl
from jax.experimental.pallas import tpu as pltpu
from jax.experimental.pallas import tpu_sc as plsc

N = 10000
E = 320000
D_IN = 128
D_HID = 256
D_OUT = 256
CB1 = 512
CB2 = 1024
PH_O = 64
COMMIT_W = 0.25

NC = 2
NS = 16
NW = NC * NS
N_PAD = 10240
SLAB = N_PAD // NS
E_PER_W = E // NW
W_FULL = 128
N_WIN = E_PER_W // W_FULL
W_TAIL = E_PER_W - N_WIN * W_FULL

@functools.lru_cache(maxsize=None)
def _mesh():
    return plsc.VectorSubcoreMesh(core_axis_name="c", subcore_axis_name="s",
                                  num_cores=NC, num_subcores=NS)


def _wid(c, s):
    return s * NC + c


def _deg_body(src_hbm, dst_hbm, zeros1, ones_hbm, out, src_v, dst_v, ones_v,
              src_t, dst_t, ones_t, acc_o, acc_i):
    c = lax.axis_index("c")
    s = lax.axis_index("s")
    w = _wid(c, s)
    pltpu.sync_copy(zeros1.at[pl.ds(s * SLAB, SLAB)], acc_o.at[pl.ds(s * SLAB, SLAB)])
    pltpu.sync_copy(zeros1.at[pl.ds(s * SLAB, SLAB)], acc_i.at[pl.ds(s * SLAB, SLAB)])
    pltpu.sync_copy(ones_hbm, ones_v)
    pltpu.sync_copy(ones_hbm.at[pl.ds(0, W_TAIL)], ones_t)
    plsc.subcore_barrier()

    e0 = w * E_PER_W

    def body(i, carry):
        base = e0 + i * W_FULL
        pltpu.sync_copy(src_hbm.at[pl.ds(base, W_FULL)], src_v)
        pltpu.sync_copy(dst_hbm.at[pl.ds(base, W_FULL)], dst_v)
        pltpu.sync_copy(ones_v, acc_o.at[src_v], add=True)
        pltpu.sync_copy(ones_v, acc_i.at[dst_v], add=True)
        return carry

    lax.fori_loop(0, N_WIN, body, 0)
    base = e0 + N_WIN * W_FULL
    pltpu.sync_copy(src_hbm.at[pl.ds(base, W_TAIL)], src_t)
    pltpu.sync_copy(dst_hbm.at[pl.ds(base, W_TAIL)], dst_t)
    pltpu.sync_copy(ones_t, acc_o.at[src_t], add=True)
    pltpu.sync_copy(ones_t, acc_i.at[dst_t], add=True)

    plsc.subcore_barrier()
    pltpu.sync_copy(acc_o.at[pl.ds(s * SLAB, SLAB)], out.at[c, 0, pl.ds(s * SLAB, SLAB)])
    pltpu.sync_copy(acc_i.at[pl.ds(s * SLAB, SLAB)], out.at[c, 1, pl.ds(s * SLAB, SLAB)])


@functools.lru_cache(maxsize=None)
def _deg_kernel():
    return pl.kernel(
        _deg_body,
        out_type=jax.ShapeDtypeStruct((NC, 2, N_PAD), jnp.float32),
        mesh=_mesh(),
        scratch_types=[
            pltpu.VMEM((W_FULL,), jnp.int32),
            pltpu.VMEM((W_FULL,), jnp.int32),
            pltpu.VMEM((W_FULL,), jnp.float32),
            pltpu.VMEM((W_TAIL,), jnp.int32),
            pltpu.VMEM((W_TAIL,), jnp.int32),
            pltpu.VMEM((W_TAIL,), jnp.float32),
            pltpu.VMEM_SHARED((N_PAD,), jnp.float32),
            pltpu.VMEM_SHARED((N_PAD,), jnp.float32),
        ],
    )


def _deg_call(src, dst, zeros1, ones):
    return _deg_kernel()(src, dst, zeros1, ones)


def _agg_body(table, src_hbm, dst_hbm, zeros2, out, src_v, dst_v, src_t,
              dst_t, rows_v, rows_t, acc, sem):
    c = lax.axis_index("c")
    s = lax.axis_index("s")
    w = _wid(c, s)
    pltpu.sync_copy(zeros2.at[pl.ds(s * SLAB, SLAB)], acc.at[pl.ds(s * SLAB, SLAB)])
    plsc.subcore_barrier()

    e0 = w * E_PER_W

    def body(i, carry):
        base = e0 + i * W_FULL
        pltpu.sync_copy(src_hbm.at[pl.ds(base, W_FULL)], src_v)
        pltpu.sync_copy(dst_hbm.at[pl.ds(base, W_FULL)], dst_v)
        pltpu.async_copy(table.at[src_v], rows_v, sem).wait()
        pltpu.sync_copy(rows_v, acc.at[dst_v], add=True)
        return carry

    lax.fori_loop(0, N_WIN, body, 0)
    base = e0 + N_WIN * W_FULL
    pltpu.sync_copy(src_hbm.at[pl.ds(base, W_TAIL)], src_t)
    pltpu.sync_copy(dst_hbm.at[pl.ds(base, W_TAIL)], dst_t)
    pltpu.async_copy(table.at[src_t], rows_t, sem).wait()
    pltpu.sync_copy(rows_t, acc.at[dst_t], add=True)

    plsc.subcore_barrier()
    pltpu.sync_copy(acc.at[pl.ds(s * SLAB, SLAB)], out.at[c, pl.ds(s * SLAB, SLAB)])


@functools.lru_cache(maxsize=None)
def _agg_kernel():
    return pl.kernel(
        _agg_body,
        out_type=jax.ShapeDtypeStruct((NC, N_PAD, D_IN), jnp.float32),
        mesh=_mesh(),
        scratch_types=[
            pltpu.VMEM((W_FULL,), jnp.int32),
            pltpu.VMEM((W_FULL,), jnp.int32),
            pltpu.VMEM((W_TAIL,), jnp.int32),
            pltpu.VMEM((W_TAIL,), jnp.int32),
            pltpu.VMEM((W_FULL, D_IN), jnp.float32),
            pltpu.VMEM((W_TAIL, D_IN), jnp.float32),
            pltpu.VMEM_SHARED((N_PAD, D_IN), jnp.float32),
            pltpu.SemaphoreType.DMA,
        ],
    )


def _agg_call(table, src, dst, zeros2):
    return _agg_kernel()(table, src, dst, zeros2)


_R_PER_W = N_PAD // NW
_GW = 80


def _prj_body(t1, t2, i1, i2, o1, o2, idx_v, rows_v, sem):
    c = lax.axis_index("c")
    s = lax.axis_index("s")
    w = _wid(c, s)
    r0 = w * _R_PER_W

    def body(i, carry):
        base = r0 + i * _GW
        pltpu.sync_copy(i1.at[pl.ds(base, _GW)], idx_v)
        pltpu.async_copy(t1.at[idx_v], rows_v, sem).wait()
        pltpu.sync_copy(rows_v, o1.at[pl.ds(base, _GW)])
        pltpu.sync_copy(i2.at[pl.ds(base, _GW)], idx_v)
        pltpu.async_copy(t2.at[idx_v], rows_v, sem).wait()
        pltpu.sync_copy(rows_v, o2.at[pl.ds(base, _GW)])
        return carry

    lax.fori_loop(0, _R_PER_W // _GW, body, 0)


@functools.lru_cache(maxsize=None)
def _prj_kernel():
    return pl.kernel(
        _prj_body,
        out_type=[jax.ShapeDtypeStruct((N_PAD, D_IN), jnp.float32),
                  jax.ShapeDtypeStruct((N_PAD, D_IN), jnp.float32)],
        mesh=_mesh(),
        scratch_types=[
            pltpu.VMEM((_GW,), jnp.int32),
            pltpu.VMEM((_GW, D_IN), jnp.float32),
            pltpu.SemaphoreType.DMA,
        ],
    )


def _prj_call(t1, t2, i1, i2):
    return _prj_kernel()(t1, t2, i1, i2)


_BLK = 1024
_NB = N_PAD // _BLK


def _mlp1_kernel(p_ref, si_ref, so_ref, w1_ref, b1_ref, lo_ref, hi_ref):
    a = (p_ref[0] + p_ref[1]) * si_ref[...]
    h = jnp.maximum(
        jax.lax.dot_general(a, w1_ref[...], (((1,), (0,)), ((), ())),
                            preferred_element_type=jnp.float32) + b1_ref[...],
        0.0)
    hs = h * so_ref[...]
    lo_ref[...] = hs[:, :D_IN]
    hi_ref[...] = hs[:, D_IN:]


def _mlp1(parts, s_in, s_out, W1, b1):
    return pl.pallas_call(
        _mlp1_kernel,
        grid=(_NB,),
        in_specs=[
            pl.BlockSpec((NC, _BLK, D_IN), lambda i: (0, i, 0)),
            pl.BlockSpec((_BLK, 1), lambda i: (i, 0)),
            pl.BlockSpec((_BLK, 1), lambda i: (i, 0)),
            pl.BlockSpec((D_IN, D_HID), lambda i: (0, 0)),
            pl.BlockSpec((1, D_HID), lambda i: (0, 0)),
        ],
        out_specs=[
            pl.BlockSpec((_BLK, D_IN), lambda i: (i, 0)),
            pl.BlockSpec((_BLK, D_IN), lambda i: (i, 0)),
        ],
        out_shape=[jax.ShapeDtypeStruct((N_PAD, D_IN), jnp.float32),
                   jax.ShapeDtypeStruct((N_PAD, D_IN), jnp.float32)],
    )(parts, s_in, s_out, W1, b1)


def _cbt_kernel(cbn1_ref, cbn2_ref, w11, b11, w12, b12, w21, b21, w22, b22,
                t1_ref, t2_ref):
    cbn1 = cbn1_ref[...]
    cbn2 = cbn2_ref[...]

    def mlp(q, wa, ba, wb, bb):
        h = jnp.maximum(
            jax.lax.dot_general(q, wa[...], (((1,), (0,)), ((), ())),
                                preferred_element_type=jnp.float32) + ba[...], 0.0)
        return (jax.lax.dot_general(h, wb[...], (((1,), (0,)), ((), ())),
                                    preferred_element_type=jnp.float32) + bb[...])

    z1 = jnp.zeros((CB1, D_IN - PH_O), jnp.float32)
    z2 = jnp.zeros((CB2, D_IN - PH_O), jnp.float32)
    t1_ref[...] = jnp.concatenate([mlp(cbn1, w11, b11, w12, b12), z1], axis=1)
    t2_ref[...] = jnp.concatenate([mlp(cbn2, w21, b21, w22, b22), z2], axis=1)


def _cb_tables(cbn1, cbn2, p1w1, p1b1, p1w2, p1b2, p2w1, p2b1, p2w2, p2b2):
    return pl.pallas_call(
        _cbt_kernel,
        out_shape=[jax.ShapeDtypeStruct((CB1, D_IN), jnp.float32),
                   jax.ShapeDtypeStruct((CB2, D_IN), jnp.float32)],
    )(cbn1, cbn2, p1w1, p1b1.reshape(1, -1), p1w2, p1b2.reshape(1, -1),
      p2w1, p2b1.reshape(1, -1), p2w2, p2b2.reshape(1, -1))


def _h2_kernel(lo_p, hi_p, si_ref, w2_ref, b2_ref, h2_ref):
    lo = (lo_p[0] + lo_p[1]) * si_ref[...]
    hi = (hi_p[0] + hi_p[1]) * si_ref[...]
    w2 = w2_ref[...]
    h2_ref[...] = (jax.lax.dot_general(lo, w2[:D_IN], (((1,), (0,)), ((), ())),
                                       preferred_element_type=jnp.float32)
                   + jax.lax.dot_general(hi, w2[D_IN:], (((1,), (0,)), ((), ())),
                                         preferred_element_type=jnp.float32)
                   + b2_ref[...])


def _h2_stage(lo_parts, hi_parts, s_in, W2, b2):
    return pl.pallas_call(
        _h2_kernel,
        grid=(_NB,),
        in_specs=[
            pl.BlockSpec((NC, _BLK, D_IN), lambda i: (0, i, 0)),
            pl.BlockSpec((NC, _BLK, D_IN), lambda i: (0, i, 0)),
            pl.BlockSpec((_BLK, 1), lambda i: (i, 0)),
            pl.BlockSpec((D_OUT, D_OUT), lambda i: (0, 0)),
            pl.BlockSpec((1, D_OUT), lambda i: (0, 0)),
        ],
        out_specs=pl.BlockSpec((_BLK, D_OUT), lambda i: (i, 0)),
        out_shape=jax.ShapeDtypeStruct((N_PAD, D_OUT), jnp.float32),
    )(lo_parts, hi_parts, s_in, W2, b2)


def _vq_kernel(hn_ref, nv_ref, cbn1_ref, cbn2_ref,
               i1_ref, i2_ref, c1_ref, c2_ref):
    i = pl.program_id(0)
    hn = hn_ref[...]
    nr_col = nv_ref[...]

    def vq(cbn, ncb):
        sim = jax.lax.dot_general(hn, cbn, (((1,), (1,)), ((), ())),
                                  preferred_element_type=jnp.float32)
        m = jnp.max(sim, axis=1, keepdims=True)
        col = jax.lax.broadcasted_iota(jnp.int32, sim.shape, 1)
        idx = jnp.min(jnp.where(sim == m, col, ncb), axis=1)
        return m[:, 0], idx

    m1, idx1 = vq(cbn1_ref[...], CB1)
    m2, idx2 = vq(cbn2_ref[...], CB2)
    i1_ref[...] = idx1.reshape(1, 1, _BLK)
    i2_ref[...] = idx2.reshape(1, 1, _BLK)

    row = jax.lax.broadcasted_iota(jnp.int32, (_BLK,), 0) + i * _BLK
    valid = row < N
    nr = nr_col[:, 0]
    ss = nr * nr
    part1 = jnp.sum(jnp.where(valid, 1.0 - 2.0 * m1 * nr + ss, 0.0))
    part2 = jnp.sum(jnp.where(valid, 1.0 - 2.0 * m2 * nr + ss, 0.0))

    @pl.when(i == 0)
    def _():
        c1_ref[...] = jnp.zeros_like(c1_ref)
        c2_ref[...] = jnp.zeros_like(c2_ref)

    c1_ref[...] = c1_ref[...] + part1
    c2_ref[...] = c2_ref[...] + part2


def _vq_stage(hn, nv, cbn1, cbn2):
    return pl.pallas_call(
        _vq_kernel,
        grid=(_NB,),
        in_specs=[
            pl.BlockSpec((_BLK, D_OUT), lambda i: (i, 0)),
            pl.BlockSpec((_BLK, 1), lambda i: (i, 0)),
            pl.BlockSpec((CB1, D_OUT), lambda i: (0, 0)),
            pl.BlockSpec((CB2, D_OUT), lambda i: (0, 0)),
        ],
        out_specs=[
            pl.BlockSpec((1, 1, _BLK), lambda i: (i, 0, 0)),
            pl.BlockSpec((1, 1, _BLK), lambda i: (i, 0, 0)),
            pl.BlockSpec((1, 1), lambda i: (0, 0)),
            pl.BlockSpec((1, 1), lambda i: (0, 0)),
        ],
        out_shape=[jax.ShapeDtypeStruct((_NB, 1, _BLK), jnp.int32),
                   jax.ShapeDtypeStruct((_NB, 1, _BLK), jnp.int32),
                   jax.ShapeDtypeStruct((1, 1), jnp.float32),
                   jax.ShapeDtypeStruct((1, 1), jnp.float32)],
    )(hn, nv, cbn1, cbn2)


def kernel(feats, edge_index, W1, b1, W2, b2, cb1, cb2,
           p1w1, p1b1, p1w2, p1b2, p2w1, p2b1, p2w2, p2b2):
    zeros1 = jnp.zeros((N_PAD,), jnp.float32)
    zeros2 = jnp.zeros((N_PAD, D_IN), jnp.float32)
    ones = jnp.ones((W_FULL,), jnp.float32)

    src = edge_index[0]
    dst = edge_index[1]
    deg = _deg_call(src, dst, zeros1, ones)
    sdst, ssrc = jax.lax.sort((dst, src), num_keys=1, is_stable=True)
    deg_out = jnp.clip(deg[0, 0] + deg[1, 0], 1.0)
    deg_in = jnp.clip(deg[0, 1] + deg[1, 1], 1.0)
    s_out = deg_out ** -0.5
    s_in = deg_in ** -0.5

    cbn1 = cb1 / (jnp.linalg.norm(cb1, axis=-1, keepdims=True) + 1e-12)
    cbn2 = cb2 / (jnp.linalg.norm(cb2, axis=-1, keepdims=True) + 1e-12)
    t1, t2 = _cb_tables(cbn1, cbn2, p1w1, p1b1, p1w2, p1b2,
                        p2w1, p2b1, p2w2, p2b2)

    h0 = feats * s_out[:N, None]
    agg1 = _agg_call(h0, ssrc, sdst, zeros2)

    h1s = _mlp1(agg1, s_in[:, None], s_out[:, None], W1, b1.reshape(1, -1))
    lo_parts = _agg_call(h1s[0], ssrc, sdst, zeros2)
    hi_parts = _agg_call(h1s[1], ssrc, sdst, zeros2)

    h2 = _h2_stage(lo_parts, hi_parts, s_in[:, None], W2, b2.reshape(1, -1))
    nv = jnp.linalg.norm(h2, axis=-1, keepdims=True) + 1e-12
    hn = h2 / nv
    i1, i2, c1s, c2s = _vq_stage(hn, nv, cbn1, cbn2)

    o1, o2 = _prj_call(t1, t2, i1.reshape(N_PAD), i2.reshape(N_PAD))

    scale = COMMIT_W / (N * D_OUT)
    c1 = (c1s[0, 0] * scale).astype(jnp.float32)
    c2 = (c2s[0, 0] * scale).astype(jnp.float32)
    return (o1[:N, :PH_O], o2[:N, :PH_O], c1, c2)
